# baseline (speedup 1.0000x reference)
import jax
import jax.numpy as jnp
import numpy as np
from jax.experimental import pallas as pl
from jax.experimental.pallas import tpu as pltpu

D_MODEL = 4096
N_HEADS = 32
N_KV_HEADS = 8
HEAD_DIM = 128
GROUP = N_HEADS // N_KV_HEADS
B = 32
T = 4
NTOK = B * T

NSPLIT = 2
GPS = N_KV_HEADS // NSPLIT


def _attn_group_kernel(x_ref, wq_ref, wk_ref, wv_ref, wo_ref, out_ref):
    g = pl.program_id(1)
    x = x_ref[...]

    k = jnp.dot(x, wk_ref[...], preferred_element_type=jnp.float32)
    v = jnp.dot(x, wv_ref[...], preferred_element_type=jnp.float32)

    row = jax.lax.broadcasted_iota(jnp.int32, (NTOK, NTOK), 0)
    col = jax.lax.broadcasted_iota(jnp.int32, (NTOK, NTOK), 1)
    valid = (col <= row) & (col >= (row // T) * T)

    scale = jnp.float32(1.0 / np.sqrt(HEAD_DIM))
    acc = jnp.zeros((NTOK, D_MODEL), jnp.float32)
    for h in range(GROUP):
        qh = jnp.dot(
            x,
            wq_ref[:, h * HEAD_DIM:(h + 1) * HEAD_DIM],
            preferred_element_type=jnp.float32,
        )
        s = jax.lax.dot_general(
            qh, k, (((1,), (1,)), ((), ())),
            preferred_element_type=jnp.float32,
        ) * scale
        s = jnp.where(valid, s, jnp.float32(-1e30))
        m = jnp.max(s, axis=1, keepdims=True)
        p = jnp.exp(s - m)
        p = p / jnp.sum(p, axis=1, keepdims=True)
        oh = jnp.dot(p, v, preferred_element_type=jnp.float32)
        acc += jnp.dot(
            oh,
            wo_ref[h * HEAD_DIM:(h + 1) * HEAD_DIM, :],
            preferred_element_type=jnp.float32,
        )

    @pl.when(g == 0)
    def _init():
        out_ref[...] = acc[None]

    @pl.when(g > 0)
    def _accum():
        out_ref[...] += acc[None]


@jax.jit
def kernel(x, Wq, Wk, Wv, Wo):
    Bx, Tx, Dx = x.shape
    xf = x.reshape(Bx * Tx, Dx)
    partial = pl.pallas_call(
        _attn_group_kernel,
        grid=(NSPLIT, GPS),
        in_specs=[
            pl.BlockSpec((NTOK, D_MODEL), lambda c, g: (0, 0)),
            pl.BlockSpec((D_MODEL, GROUP * HEAD_DIM),
                         lambda c, g: (0, c * GPS + g)),
            pl.BlockSpec((D_MODEL, HEAD_DIM), lambda c, g: (0, c * GPS + g)),
            pl.BlockSpec((D_MODEL, HEAD_DIM), lambda c, g: (0, c * GPS + g)),
            pl.BlockSpec((GROUP * HEAD_DIM, D_MODEL),
                         lambda c, g: (c * GPS + g, 0)),
        ],
        out_specs=pl.BlockSpec((1, NTOK, D_MODEL), lambda c, g: (c, 0, 0)),
        out_shape=jax.ShapeDtypeStruct((NSPLIT, NTOK, D_MODEL), jnp.float32),
        compiler_params=pltpu.CompilerParams(
            dimension_semantics=("parallel", "arbitrary"),
        ),
    )(xf, Wq, Wk, Wv, Wo)
    return (partial[0] + partial[1]).reshape(Bx, Tx, Dx)
